# baseline (speedup 1.0000x reference)
import math

import jax
import jax.numpy as jnp
import numpy as np
from jax.experimental import pallas as pl
from jax.experimental.pallas import tpu as pltpu

ENC = 256
M = 4
H = 32
W = 32
PE = 64
KS = 11
K = KS * KS
KP = 128
DENOM = 2.0 * (KS / 3.0) ** 2

_BINS = [(0, 11), (10, 22), (21, 32)]


def _body(feat_ref, cell_ref, gaze_ref,
          w1t_ref, b1_ref, w2t_ref, b2_ref, wvt_ref, bv_ref,
          ws1t_ref, bs1_ref, ws2_ref, consts_ref,
          w1s_ref, cb1_ref, w2s_ref, cb2_ref, woutt_ref, bout_ref,
          out_ref, upd_ref):
    Bb = feat_ref.shape[0]
    f = feat_ref[...]
    gz = gaze_ref[...]
    gxn = gz[:, 0:1]
    gyn = gz[:, 1:2]
    div = consts_ref[0:1, 0:16]

    ax = gxn * div
    ay = gyn * div
    pe = jnp.concatenate([jnp.sin(ax), jnp.cos(ax), jnp.sin(ay), jnp.cos(ay)], axis=1)
    g = jnp.concatenate([f, pe], axis=1)

    bf = jnp.bfloat16
    h1 = jnp.maximum(jnp.dot(g.astype(bf), w1t_ref[...], preferred_element_type=jnp.float32) + b1_ref[...], 0.0)
    h2 = jnp.maximum(jnp.dot(h1.astype(bf), w2t_ref[...], preferred_element_type=jnp.float32) + b2_ref[...], 0.0)
    wv = (jnp.dot(h2.astype(bf), wvt_ref[...], preferred_element_type=jnp.float32) + bv_ref[...])
    wv = wv.reshape(Bb, KP, M)

    ox = consts_ref[1:2, :]
    oy = consts_ref[2:3, :]
    kmask = consts_ref[3:4, :]
    gx = jnp.clip(gxn * (W - 1.0), 0.0, W - 1.0)
    gy = jnp.clip(gyn * (H - 1.0), 0.0, H - 1.0)
    x0 = jnp.floor(gx)
    y0 = jnp.floor(gy)
    xs = jnp.clip(x0 + ox, 0.0, W - 1.0)
    ys = jnp.clip(y0 + oy, 0.0, H - 1.0)
    dx = xs - gx
    dy = ys - gy
    gauss = jnp.exp(-(dx * dx + dy * dy) / DENOM) * kmask
    nw = gauss / jnp.maximum(jnp.sum(gauss, axis=1, keepdims=True), 1e-8)
    idx = (ys * W + xs).astype(jnp.int32)

    iota = jax.lax.broadcasted_iota(jnp.int32, (Bb, KP, H * W), 2)
    onehot = (idx[:, :, None] == iota).astype(jnp.float32)

    cell = cell_ref[...]
    cf = cell.reshape(Bb, M, H * W)

    prev = jax.lax.dot_general(onehot, cf,
                               (((2,), (2,)), ((0,), (0,))),
                               preferred_element_type=jnp.float32)

    gi = jnp.concatenate([prev, wv], axis=-1).reshape(Bb * KP, 2 * M)
    t = jnp.maximum(jnp.dot(gi, ws1t_ref[...], preferred_element_type=jnp.float32) + bs1_ref[...], 0.0)
    s = jnp.sum(t * ws2_ref[...], axis=1, keepdims=True) + consts_ref[4:5, 0:1]
    gate = jax.nn.sigmoid(s).reshape(Bb, KP, 1)

    w_all = nw[:, :, None] * gate
    newv = (1.0 - w_all) * prev + w_all * wv
    svals = jnp.concatenate([w_all * newv, w_all], axis=-1)

    scat = jax.lax.dot_general(svals, onehot,
                               (((1,), (1,)), ((0,), (0,))),
                               preferred_element_type=jnp.float32)
    den = scat[:, 4:5, :]
    keep = 1.0 - jnp.minimum(den, 1.0)
    upd = keep * cf + scat[:, 0:4, :]
    upd_ref[...] = upd.reshape(Bb, M, H, W)

    xcl = jnp.transpose(upd, (0, 2, 1)).reshape(Bb, H, W, M).astype(bf)
    xp = jnp.pad(xcl, ((0, 0), (1, 1), (1, 1), (0, 0)))
    acc = None
    for kh in range(3):
        for kw in range(3):
            sl = xp[:, kh:kh + H, kw:kw + W, :].reshape(Bb * H * W, M)
            part = jnp.dot(sl, w1s_ref[kh * 3 + kw], preferred_element_type=jnp.float32)
            acc = part if acc is None else acc + part
    y1 = jnp.maximum(acc + cb1_ref[...], 0.0).astype(bf)
    y1p = jnp.pad(y1.reshape(Bb, H, W, 128), ((0, 0), (1, 1), (1, 1), (0, 0)))
    acc2 = None
    for kh in range(3):
        for kw in range(3):
            sl = y1p[:, kh:kh + H, kw:kw + W, :].reshape(Bb * H * W, 128)
            part = jnp.dot(sl, w2s_ref[kh * 3 + kw], preferred_element_type=jnp.float32)
            acc2 = part if acc2 is None else acc2 + part
    y2 = jnp.maximum(acc2 + cb2_ref[...], 0.0).reshape(Bb, H, W, 64)

    cols = []
    for (w0, w1) in _BINS:
        cols.append(jnp.sum(y2[:, :, w0:w1, :], axis=2))
    rows = []
    for (h0, h1) in _BINS:
        for j, (w0, w1) in enumerate(_BINS):
            seg = jnp.sum(cols[j][:, h0:h1, :], axis=1)
            rows.append(seg * (1.0 / ((h1 - h0) * (w1 - w0))))
    pflat = jnp.concatenate(rows, axis=1)

    out_ref[...] = jnp.dot(pflat.astype(bf), woutt_ref[...], preferred_element_type=jnp.float32) + bout_ref[...]


def kernel(features, cell_state, gaze_coords, W1, b1, W2, b2, Wv, bv,
           Ws1, bs1, Ws2, bs2, cw1, cb1, cw2, cb2, Wout, bout):
    B = features.shape[0]
    Bb = 8 if B % 8 == 0 else 1
    f32 = jnp.float32

    half = PE // 2
    div = np.exp(np.arange(0, half, 2, dtype=np.float32) * (-(math.log(10000.0) / half)))
    r = KS // 2
    offs = np.arange(-r, r + 1)
    oxg, oyg = np.meshgrid(offs, offs, indexing='xy')
    consts = np.zeros((8, 128), dtype=np.float32)
    consts[0, :16] = div
    consts[1, :K] = oxg.reshape(-1)
    consts[2, :K] = oyg.reshape(-1)
    consts[3, :K] = 1.0
    consts_j = jnp.asarray(consts)
    consts_j = consts_j.at[4, 0].set(jnp.asarray(bs2).reshape(-1)[0])

    perm = np.concatenate([
        np.arange(ENC),
        ENC + 2 * np.arange(16), ENC + 1 + 2 * np.arange(16),
        ENC + 32 + 2 * np.arange(16), ENC + 33 + 2 * np.arange(16),
    ])
    bf = jnp.bfloat16
    W1pT = W1[:, perm].T.astype(bf)
    W2T = W2.T.astype(bf)
    WvT = jnp.pad(Wv.reshape(K, M, 128), ((0, KP - K), (0, 0), (0, 0))).reshape(KP * M, 128).T.astype(bf)
    bvr = jnp.pad(bv.reshape(K, M), ((0, KP - K), (0, 0))).reshape(1, KP * M)
    Ws1T = Ws1.T.astype(f32)
    w1s = cw1.transpose(2, 3, 1, 0).reshape(9, M, 128).astype(bf)
    w2s = cw2.transpose(2, 3, 1, 0).reshape(9, 128, 64).astype(bf)
    WoutT2 = Wout.reshape(576, 64, 9).transpose(0, 2, 1).reshape(576, 576).T.astype(bf)
    gaze_pad = jnp.pad(gaze_coords.astype(f32), ((0, 0), (0, 126)))

    b1r = b1.reshape(1, 256)
    b2r = b2.reshape(1, 128)
    bs1r = bs1.reshape(1, 64)
    ws2r = Ws2.reshape(1, 64)
    cb1r = cb1.reshape(1, 128)
    cb2r = cb2.reshape(1, 64)
    boutr = bout.reshape(1, 576)

    grid = (B // Bb,)

    def fixed(shape):
        nd = len(shape)
        return pl.BlockSpec(shape, lambda i, _n=nd: (0,) * _n)

    out, updated = pl.pallas_call(
        _body,
        grid=grid,
        in_specs=[
            pl.BlockSpec((Bb, ENC), lambda i: (i, 0)),
            pl.BlockSpec((Bb, M, H, W), lambda i: (i, 0, 0, 0)),
            pl.BlockSpec((Bb, 128), lambda i: (i, 0)),
            fixed((ENC + PE, 256)),
            fixed((1, 256)),
            fixed((256, 128)),
            fixed((1, 128)),
            fixed((128, KP * M)),
            fixed((1, KP * M)),
            fixed((2 * M, 64)),
            fixed((1, 64)),
            fixed((1, 64)),
            fixed((8, 128)),
            fixed((9, M, 128)),
            fixed((1, 128)),
            fixed((9, 128, 64)),
            fixed((1, 64)),
            fixed((576, 576)),
            fixed((1, 576)),
        ],
        out_specs=[
            pl.BlockSpec((Bb, 576), lambda i: (i, 0)),
            pl.BlockSpec((Bb, M, H, W), lambda i: (i, 0, 0, 0)),
        ],
        out_shape=[
            jax.ShapeDtypeStruct((B, 576), f32),
            jax.ShapeDtypeStruct((B, M, H, W), f32),
        ],
        compiler_params=pltpu.CompilerParams(
            dimension_semantics=("arbitrary",),
        ),
    )(features.astype(f32), cell_state.astype(f32), gaze_pad,
      W1pT, b1r, W2T, b2r, WvT, bvr, Ws1T, bs1r, ws2r, consts_j,
      w1s, cb1r, w2s, cb2r, WoutT2, boutr)
    return out, updated


# speedup vs baseline: 1.6820x; 1.6820x over previous
import math

import jax
import jax.numpy as jnp
import numpy as np
from jax.experimental import pallas as pl
from jax.experimental.pallas import tpu as pltpu

ENC = 256
M = 4
H = 32
W = 32
PE = 64
KS = 11
K = KS * KS
KP = 128
DENOM = 2.0 * (KS / 3.0) ** 2

_BINS = [(0, 11), (10, 22), (21, 32)]


def _body(feat_ref, cell_ref, gaze_ref,
          w1t_ref, b1_ref, w2t_ref, b2_ref, wvt_ref, bv_ref,
          ws1t_ref, bs1_ref, ws2_ref, consts_ref,
          w1s_ref, cb1_ref, w2s_ref, cb2_ref, woutt_ref, bout_ref,
          out_ref, upd_ref):
    Bb = feat_ref.shape[0]
    f = feat_ref[...]
    gz = gaze_ref[...]
    gxn = gz[:, 0:1]
    gyn = gz[:, 1:2]
    div = consts_ref[0:1, 0:16]

    ax = gxn * div
    ay = gyn * div
    pe = jnp.concatenate([jnp.sin(ax), jnp.cos(ax), jnp.sin(ay), jnp.cos(ay)], axis=1)
    g = jnp.concatenate([f, pe], axis=1)

    bf = jnp.bfloat16
    h1 = jnp.maximum(jnp.dot(g.astype(bf), w1t_ref[...], preferred_element_type=jnp.float32) + b1_ref[...], 0.0)
    h2 = jnp.maximum(jnp.dot(h1.astype(bf), w2t_ref[...], preferred_element_type=jnp.float32) + b2_ref[...], 0.0)
    wv = (jnp.dot(h2.astype(bf), wvt_ref[...], preferred_element_type=jnp.float32) + bv_ref[...])
    wv = wv.reshape(Bb, KP, M)

    ox = consts_ref[1:2, :]
    oy = consts_ref[2:3, :]
    kmask = consts_ref[3:4, :]
    gx = jnp.clip(gxn * (W - 1.0), 0.0, W - 1.0)
    gy = jnp.clip(gyn * (H - 1.0), 0.0, H - 1.0)
    x0 = jnp.floor(gx)
    y0 = jnp.floor(gy)
    xs = jnp.clip(x0 + ox, 0.0, W - 1.0)
    ys = jnp.clip(y0 + oy, 0.0, H - 1.0)
    dx = xs - gx
    dy = ys - gy
    gauss = jnp.exp(-(dx * dx + dy * dy) / DENOM) * kmask
    nw = gauss / jnp.maximum(jnp.sum(gauss, axis=1, keepdims=True), 1e-8)
    idx = (ys * W + xs).astype(jnp.int32)

    iota = jax.lax.broadcasted_iota(jnp.int32, (Bb, KP, H * W), 2)
    onehot = (idx[:, :, None] == iota).astype(jnp.float32)

    cell = cell_ref[...]
    cf = cell.reshape(Bb, M, H * W)

    prev = jax.lax.dot_general(onehot, cf,
                               (((2,), (2,)), ((0,), (0,))),
                               preferred_element_type=jnp.float32)

    gi = jnp.concatenate([prev, wv], axis=-1).reshape(Bb * KP, 2 * M)
    t = jnp.maximum(jnp.dot(gi, ws1t_ref[...], preferred_element_type=jnp.float32) + bs1_ref[...], 0.0)
    s = jnp.sum(t * ws2_ref[...], axis=1, keepdims=True) + consts_ref[4:5, 0:1]
    gate = jax.nn.sigmoid(s).reshape(Bb, KP, 1)

    w_all = nw[:, :, None] * gate
    newv = (1.0 - w_all) * prev + w_all * wv
    svals = jnp.concatenate([w_all * newv, w_all], axis=-1)

    scat = jax.lax.dot_general(svals, onehot,
                               (((1,), (1,)), ((0,), (0,))),
                               preferred_element_type=jnp.float32)
    den = scat[:, 4:5, :]
    keep = 1.0 - jnp.minimum(den, 1.0)
    upd = keep * cf + scat[:, 0:4, :]
    upd_ref[...] = upd.reshape(Bb, M, H, W)

    xcl = jnp.transpose(upd, (0, 2, 1)).reshape(Bb, H, W, M).astype(bf)
    xp = jnp.pad(xcl, ((0, 0), (1, 1), (1, 1), (0, 0)))
    acc = None
    for kw in range(3):
        xw = xp[:, :, kw:kw + W, :]
        sl = jnp.concatenate(
            [xw[:, kh:kh + H, :, :] for kh in range(3)], axis=-1
        ).reshape(Bb * H * W, 3 * M)
        part = jnp.dot(sl, w1s_ref[kw], preferred_element_type=jnp.float32)
        acc = part if acc is None else acc + part
    y1 = jnp.maximum(acc + cb1_ref[...], 0.0).astype(bf)
    y1p = jnp.pad(y1.reshape(Bb, H, W, 128), ((0, 0), (1, 1), (1, 1), (0, 0)))
    acc2 = None
    for kw in range(3):
        yw = y1p[:, :, kw:kw + W, :]
        sl = jnp.concatenate(
            [yw[:, kh:kh + H, :, :] for kh in range(3)], axis=-1
        ).reshape(Bb * H * W, 3 * 128)
        part = jnp.dot(sl, w2s_ref[kw], preferred_element_type=jnp.float32)
        acc2 = part if acc2 is None else acc2 + part
    y2 = jnp.maximum(acc2 + cb2_ref[...], 0.0).reshape(Bb, H, W, 64)

    rows = []
    for (h0, h1) in _BINS:
        rows.append(jnp.sum(y2[:, h0:h1, :, :], axis=1))
    segs = []
    for i, (h0, h1) in enumerate(_BINS):
        for (w0, w1) in _BINS:
            seg = jnp.sum(rows[i][:, w0:w1, :], axis=1)
            segs.append(seg * (1.0 / ((h1 - h0) * (w1 - w0))))
    pflat = jnp.concatenate(segs, axis=1)

    out_ref[...] = jnp.dot(pflat.astype(bf), woutt_ref[...], preferred_element_type=jnp.float32) + bout_ref[...]


def kernel(features, cell_state, gaze_coords, W1, b1, W2, b2, Wv, bv,
           Ws1, bs1, Ws2, bs2, cw1, cb1, cw2, cb2, Wout, bout):
    B = features.shape[0]
    Bb = 8 if B % 8 == 0 else 1
    f32 = jnp.float32

    half = PE // 2
    div = np.exp(np.arange(0, half, 2, dtype=np.float32) * (-(math.log(10000.0) / half)))
    r = KS // 2
    offs = np.arange(-r, r + 1)
    oxg, oyg = np.meshgrid(offs, offs, indexing='xy')
    consts = np.zeros((8, 128), dtype=np.float32)
    consts[0, :16] = div
    consts[1, :K] = oxg.reshape(-1)
    consts[2, :K] = oyg.reshape(-1)
    consts[3, :K] = 1.0
    consts_j = jnp.asarray(consts)
    consts_j = consts_j.at[4, 0].set(jnp.asarray(bs2).reshape(-1)[0])

    perm = np.concatenate([
        np.arange(ENC),
        ENC + 2 * np.arange(16), ENC + 1 + 2 * np.arange(16),
        ENC + 32 + 2 * np.arange(16), ENC + 33 + 2 * np.arange(16),
    ])
    bf = jnp.bfloat16
    W1pT = W1[:, perm].T.astype(bf)
    W2T = W2.T.astype(bf)
    WvT = jnp.pad(Wv.reshape(K, M, 128), ((0, KP - K), (0, 0), (0, 0))).reshape(KP * M, 128).T.astype(bf)
    bvr = jnp.pad(bv.reshape(K, M), ((0, KP - K), (0, 0))).reshape(1, KP * M)
    Ws1T = Ws1.T.astype(f32)
    w1s = jnp.stack([
        jnp.concatenate([cw1[:, :, kh, kw].T for kh in range(3)], axis=0)
        for kw in range(3)]).astype(bf)
    w2s = jnp.stack([
        jnp.concatenate([cw2[:, :, kh, kw].T for kh in range(3)], axis=0)
        for kw in range(3)]).astype(bf)
    WoutT2 = Wout.reshape(576, 64, 9).transpose(0, 2, 1).reshape(576, 576).T.astype(bf)
    gaze_pad = jnp.pad(gaze_coords.astype(f32), ((0, 0), (0, 126)))

    b1r = b1.reshape(1, 256)
    b2r = b2.reshape(1, 128)
    bs1r = bs1.reshape(1, 64)
    ws2r = Ws2.reshape(1, 64)
    cb1r = cb1.reshape(1, 128)
    cb2r = cb2.reshape(1, 64)
    boutr = bout.reshape(1, 576)

    grid = (B // Bb,)

    def fixed(shape):
        nd = len(shape)
        return pl.BlockSpec(shape, lambda i, _n=nd: (0,) * _n)

    out, updated = pl.pallas_call(
        _body,
        grid=grid,
        in_specs=[
            pl.BlockSpec((Bb, ENC), lambda i: (i, 0)),
            pl.BlockSpec((Bb, M, H, W), lambda i: (i, 0, 0, 0)),
            pl.BlockSpec((Bb, 128), lambda i: (i, 0)),
            fixed((ENC + PE, 256)),
            fixed((1, 256)),
            fixed((256, 128)),
            fixed((1, 128)),
            fixed((128, KP * M)),
            fixed((1, KP * M)),
            fixed((2 * M, 64)),
            fixed((1, 64)),
            fixed((1, 64)),
            fixed((8, 128)),
            fixed((3, 3 * M, 128)),
            fixed((1, 128)),
            fixed((3, 3 * 128, 64)),
            fixed((1, 64)),
            fixed((576, 576)),
            fixed((1, 576)),
        ],
        out_specs=[
            pl.BlockSpec((Bb, 576), lambda i: (i, 0)),
            pl.BlockSpec((Bb, M, H, W), lambda i: (i, 0, 0, 0)),
        ],
        out_shape=[
            jax.ShapeDtypeStruct((B, 576), f32),
            jax.ShapeDtypeStruct((B, M, H, W), f32),
        ],
        compiler_params=pltpu.CompilerParams(
            dimension_semantics=("arbitrary",),
        ),
    )(features.astype(f32), cell_state.astype(f32), gaze_pad,
      W1pT, b1r, W2T, b2r, WvT, bvr, Ws1T, bs1r, ws2r, consts_j,
      w1s, cb1r, w2s, cb2r, WoutT2, boutr)
    return out, updated


# speedup vs baseline: 1.7991x; 1.0696x over previous
import math

import jax
import jax.numpy as jnp
import numpy as np
from jax.experimental import pallas as pl
from jax.experimental.pallas import tpu as pltpu

ENC = 256
M = 4
H = 32
W = 32
PE = 64
KS = 11
K = KS * KS
KP = 128
DENOM = 2.0 * (KS / 3.0) ** 2

_BINS = [(0, 11), (10, 22), (21, 32)]


def _body(feat_ref, cell_ref, gaze_ref,
          w1t_ref, b1_ref, w2t_ref, b2_ref, wvt_ref, bv_ref,
          ws1t_ref, bs1_ref, ws2_ref, consts_ref,
          w1s_ref, cb1_ref, w2s_ref, cb2_ref, woutt_ref, bout_ref,
          out_ref, upd_ref):
    Bb = feat_ref.shape[0]
    f = feat_ref[...]
    gz = gaze_ref[...]
    gxn = gz[:, 0:1]
    gyn = gz[:, 1:2]
    div = consts_ref[0:1, 0:16]

    ax = gxn * div
    ay = gyn * div
    pe = jnp.concatenate([jnp.sin(ax), jnp.cos(ax), jnp.sin(ay), jnp.cos(ay)], axis=1)
    g = jnp.concatenate([f, pe], axis=1)

    bf = jnp.bfloat16
    h1 = jnp.maximum(jnp.dot(g.astype(bf), w1t_ref[...], preferred_element_type=jnp.float32) + b1_ref[...], 0.0)
    h2 = jnp.maximum(jnp.dot(h1.astype(bf), w2t_ref[...], preferred_element_type=jnp.float32) + b2_ref[...], 0.0)
    wv = (jnp.dot(h2.astype(bf), wvt_ref[...], preferred_element_type=jnp.float32) + bv_ref[...])
    wv = wv.reshape(Bb, KP, M)

    ox = consts_ref[1:2, :]
    oy = consts_ref[2:3, :]
    kmask = consts_ref[3:4, :]
    gx = jnp.clip(gxn * (W - 1.0), 0.0, W - 1.0)
    gy = jnp.clip(gyn * (H - 1.0), 0.0, H - 1.0)
    x0 = jnp.floor(gx)
    y0 = jnp.floor(gy)
    xs = jnp.clip(x0 + ox, 0.0, W - 1.0)
    ys = jnp.clip(y0 + oy, 0.0, H - 1.0)
    dx = xs - gx
    dy = ys - gy
    gauss = jnp.exp(-(dx * dx + dy * dy) / DENOM) * kmask
    nw = gauss / jnp.maximum(jnp.sum(gauss, axis=1, keepdims=True), 1e-8)
    idx = (ys * W + xs).astype(jnp.int32)

    iota = jax.lax.broadcasted_iota(jnp.int32, (Bb, KP, H * W), 2)
    onehot = (idx[:, :, None] == iota).astype(jnp.float32)

    cell = cell_ref[...]
    cf = cell.reshape(Bb, M, H * W)

    prev = jax.lax.dot_general(onehot, cf,
                               (((2,), (2,)), ((0,), (0,))),
                               preferred_element_type=jnp.float32)

    gi = jnp.concatenate([prev, wv], axis=-1).reshape(Bb * KP, 2 * M)
    t = jnp.maximum(jnp.dot(gi, ws1t_ref[...], preferred_element_type=jnp.float32) + bs1_ref[...], 0.0)
    s = jnp.sum(t * ws2_ref[...], axis=1, keepdims=True) + consts_ref[4:5, 0:1]
    gate = jax.nn.sigmoid(s).reshape(Bb, KP, 1)

    w_all = nw[:, :, None] * gate
    newv = (1.0 - w_all) * prev + w_all * wv
    svals = jnp.concatenate([w_all * newv, w_all], axis=-1)

    scat = jax.lax.dot_general(svals, onehot,
                               (((1,), (1,)), ((0,), (0,))),
                               preferred_element_type=jnp.float32)
    den = scat[:, 4:5, :]
    keep = 1.0 - jnp.minimum(den, 1.0)
    upd = keep * cf + scat[:, 0:4, :]
    upd_ref[...] = upd.reshape(Bb, M, H, W)

    xcl = jnp.transpose(upd, (0, 2, 1)).reshape(Bb, H, W, M).astype(bf)
    xph = jnp.pad(xcl, ((0, 0), (1, 1), (0, 0), (0, 0)))
    x12 = jnp.concatenate(
        [xph[:, kh:kh + H, :, :] for kh in range(3)], axis=-1)
    x12p = jnp.pad(x12, ((0, 0), (0, 0), (1, 1), (0, 0)))
    acc = None
    for kw in range(3):
        sl = x12p[:, :, kw:kw + W, :].reshape(Bb * H * W, 3 * M)
        part = jnp.dot(sl, w1s_ref[kw], preferred_element_type=jnp.float32)
        acc = part if acc is None else acc + part
    y1 = jnp.maximum(acc + cb1_ref[...], 0.0).astype(bf)
    y1p = jnp.pad(y1.reshape(Bb, H, W, 128), ((0, 0), (1, 1), (1, 1), (0, 0)))
    acc2 = None
    for kw in range(3):
        yw = y1p[:, :, kw:kw + W, :]
        sl = jnp.concatenate(
            [yw[:, kh:kh + H, :, :] for kh in range(3)], axis=-1
        ).reshape(Bb * H * W, 3 * 128)
        part = jnp.dot(sl, w2s_ref[kw], preferred_element_type=jnp.float32)
        acc2 = part if acc2 is None else acc2 + part
    y2 = jnp.maximum(acc2 + cb2_ref[...], 0.0).reshape(Bb, H, W, 64)

    rows = []
    for (h0, h1) in _BINS:
        rows.append(jnp.sum(y2[:, h0:h1, :, :], axis=1))
    segs = []
    for i, (h0, h1) in enumerate(_BINS):
        for (w0, w1) in _BINS:
            seg = jnp.sum(rows[i][:, w0:w1, :], axis=1)
            segs.append(seg * (1.0 / ((h1 - h0) * (w1 - w0))))
    pflat = jnp.concatenate(segs, axis=1)

    out_ref[...] = jnp.dot(pflat.astype(bf), woutt_ref[...], preferred_element_type=jnp.float32) + bout_ref[...]


def kernel(features, cell_state, gaze_coords, W1, b1, W2, b2, Wv, bv,
           Ws1, bs1, Ws2, bs2, cw1, cb1, cw2, cb2, Wout, bout):
    B = features.shape[0]
    Bb = 16 if B % 16 == 0 else 1
    f32 = jnp.float32

    half = PE // 2
    div = np.exp(np.arange(0, half, 2, dtype=np.float32) * (-(math.log(10000.0) / half)))
    r = KS // 2
    offs = np.arange(-r, r + 1)
    oxg, oyg = np.meshgrid(offs, offs, indexing='xy')
    consts = np.zeros((8, 128), dtype=np.float32)
    consts[0, :16] = div
    consts[1, :K] = oxg.reshape(-1)
    consts[2, :K] = oyg.reshape(-1)
    consts[3, :K] = 1.0
    consts_j = jnp.asarray(consts)
    consts_j = consts_j.at[4, 0].set(jnp.asarray(bs2).reshape(-1)[0])

    perm = np.concatenate([
        np.arange(ENC),
        ENC + 2 * np.arange(16), ENC + 1 + 2 * np.arange(16),
        ENC + 32 + 2 * np.arange(16), ENC + 33 + 2 * np.arange(16),
    ])
    bf = jnp.bfloat16
    W1pT = W1[:, perm].T.astype(bf)
    W2T = W2.T.astype(bf)
    WvT = jnp.pad(Wv.reshape(K, M, 128), ((0, KP - K), (0, 0), (0, 0))).reshape(KP * M, 128).T.astype(bf)
    bvr = jnp.pad(bv.reshape(K, M), ((0, KP - K), (0, 0))).reshape(1, KP * M)
    Ws1T = Ws1.T.astype(f32)
    w1s = jnp.stack([
        jnp.concatenate([cw1[:, :, kh, kw].T for kh in range(3)], axis=0)
        for kw in range(3)]).astype(bf)
    w2s = jnp.stack([
        jnp.concatenate([cw2[:, :, kh, kw].T for kh in range(3)], axis=0)
        for kw in range(3)]).astype(bf)
    WoutT2 = Wout.reshape(576, 64, 9).transpose(0, 2, 1).reshape(576, 576).T.astype(bf)
    gaze_pad = jnp.pad(gaze_coords.astype(f32), ((0, 0), (0, 126)))

    b1r = b1.reshape(1, 256)
    b2r = b2.reshape(1, 128)
    bs1r = bs1.reshape(1, 64)
    ws2r = Ws2.reshape(1, 64)
    cb1r = cb1.reshape(1, 128)
    cb2r = cb2.reshape(1, 64)
    boutr = bout.reshape(1, 576)

    grid = (B // Bb,)

    def fixed(shape):
        nd = len(shape)
        return pl.BlockSpec(shape, lambda i, _n=nd: (0,) * _n)

    out, updated = pl.pallas_call(
        _body,
        grid=grid,
        in_specs=[
            pl.BlockSpec((Bb, ENC), lambda i: (i, 0)),
            pl.BlockSpec((Bb, M, H, W), lambda i: (i, 0, 0, 0)),
            pl.BlockSpec((Bb, 128), lambda i: (i, 0)),
            fixed((ENC + PE, 256)),
            fixed((1, 256)),
            fixed((256, 128)),
            fixed((1, 128)),
            fixed((128, KP * M)),
            fixed((1, KP * M)),
            fixed((2 * M, 64)),
            fixed((1, 64)),
            fixed((1, 64)),
            fixed((8, 128)),
            fixed((3, 3 * M, 128)),
            fixed((1, 128)),
            fixed((3, 3 * 128, 64)),
            fixed((1, 64)),
            fixed((576, 576)),
            fixed((1, 576)),
        ],
        out_specs=[
            pl.BlockSpec((Bb, 576), lambda i: (i, 0)),
            pl.BlockSpec((Bb, M, H, W), lambda i: (i, 0, 0, 0)),
        ],
        out_shape=[
            jax.ShapeDtypeStruct((B, 576), f32),
            jax.ShapeDtypeStruct((B, M, H, W), f32),
        ],
        compiler_params=pltpu.CompilerParams(
            dimension_semantics=("arbitrary",),
        ),
    )(features.astype(f32), cell_state.astype(f32), gaze_pad,
      W1pT, b1r, W2T, b2r, WvT, bvr, Ws1T, bs1r, ws2r, consts_j,
      w1s, cb1r, w2s, cb2r, WoutT2, boutr)
    return out, updated
